# Initial kernel scaffold; baseline (speedup 1.0000x reference)
import jax
import jax.numpy as jnp
from jax.experimental import pallas as pl


def kernel(queries, keys):
    raise NotImplementedError("write your pallas kernel here")


# speedup vs baseline: 3.6123x; 3.6123x over previous
import functools

import jax
import jax.numpy as jnp
from jax.experimental import pallas as pl
from jax.experimental.pallas import tpu as pltpu

_K = 16
_CHUNK = 512
_IMAX = 2147483647
_IMIN = -2147483648


def _knn_body(nkeys, q_ref, k_ref, dv_ref, di_ref, d_scr, rv_scr, ri_scr):
    c = pl.program_id(0)
    nc = pl.num_programs(0)
    nq = q_ref.shape[0]
    cw = k_ref.shape[0]

    @pl.when(c == 0)
    def _init():
        rv_scr[...] = jnp.full((nq, _K), jnp.inf, dtype=jnp.float32)
        ri_scr[...] = -(jax.lax.broadcasted_iota(jnp.int32, (nq, _K), 1) + 1)

    q = q_ref[...]
    k = k_ref[...]
    qn = jnp.sum(q * q, axis=1, keepdims=True)
    kn = jnp.sum(k * k, axis=1)[None, :]
    gidx = c * cw + jax.lax.broadcasted_iota(jnp.int32, (1, cw), 1)
    kn = jnp.where(gidx < nkeys, kn, jnp.inf)
    d = -2.0 * jnp.dot(q, k.T, preferred_element_type=jnp.float32)
    d = d + qn
    d = d + kn
    d_scr[...] = d

    def _merge(_):
        dd = d_scr[...]
        rvv = rv_scr[...]
        rii = ri_scr[...]
        m = jnp.min(dd, axis=1, keepdims=True)
        thr = jnp.max(rvv, axis=1, keepdims=True)
        need = m < thr
        gb = jnp.broadcast_to(gidx, dd.shape)
        gsel = jnp.min(jnp.where(dd == m, gb, _IMAX), axis=1, keepdims=True)
        d_scr[...] = jnp.where((gb == gsel) & need, jnp.inf, dd)
        evict = jnp.max(jnp.where(rvv == thr, rii, _IMIN), axis=1,
                        keepdims=True)
        sel = (rii == evict) & need
        rv_scr[...] = jnp.where(sel, jnp.broadcast_to(m, rvv.shape), rvv)
        ri_scr[...] = jnp.where(sel, jnp.broadcast_to(gsel, rii.shape), rii)
        return jnp.max(need.astype(jnp.int32)) > 0

    jax.lax.while_loop(lambda go: go, _merge, True)

    @pl.when(c == nc - 1)
    def _finalize():
        vals = rv_scr[...]
        idxs = ri_scr[...]
        vcols = []
        icols = []
        for _ in range(_K):
            m = jnp.min(vals, axis=1, keepdims=True)
            gs = jnp.min(jnp.where(vals == m, idxs, _IMAX), axis=1,
                         keepdims=True)
            vcols.append(m)
            icols.append(gs)
            vals = jnp.where(idxs == gs, jnp.inf, vals)
        dv_ref[...] = jnp.concatenate(vcols, axis=1)
        di_ref[...] = jnp.concatenate(icols, axis=1)


def _knn(queries, keys, chunk):
    nq, dim = queries.shape
    nk = keys.shape[0]
    nc = pl.cdiv(nk, chunk)
    npad = nc * chunk - nk
    keys_p = jnp.pad(keys, ((0, npad), (0, 0))) if npad else keys
    return pl.pallas_call(
        functools.partial(_knn_body, nk),
        grid=(nc,),
        in_specs=[
            pl.BlockSpec((nq, dim), lambda c: (0, 0)),
            pl.BlockSpec((chunk, dim), lambda c: (c, 0)),
        ],
        out_specs=[
            pl.BlockSpec((nq, _K), lambda c: (0, 0)),
            pl.BlockSpec((nq, _K), lambda c: (0, 0)),
        ],
        out_shape=[
            jax.ShapeDtypeStruct((nq, _K), jnp.float32),
            jax.ShapeDtypeStruct((nq, _K), jnp.int32),
        ],
        scratch_shapes=[
            pltpu.VMEM((nq, chunk), jnp.float32),
            pltpu.VMEM((nq, _K), jnp.float32),
            pltpu.VMEM((nq, _K), jnp.int32),
        ],
        compiler_params=pltpu.CompilerParams(
            dimension_semantics=("arbitrary",)),
    )(queries, keys_p)


def kernel(queries, keys):
    dists, idx = _knn(queries, keys, _CHUNK)
    return (dists, idx)
